# baseline (speedup 1.0000x reference)
import functools

import jax
import jax.numpy as jnp
import numpy as np
from jax import lax
from jax.experimental import pallas as pl
from jax.experimental.pallas import tpu as pltpu
from jax.experimental.pallas import tpu_sc as plsc

HIDDEN = 128
INPUT_DIM = 768
HL = 64
NB = 8
T = 128
N = 512
E = 100000
EPAD = 100352
NTILES = 16
CHUNK = EPAD // NTILES
ROWS = CHUNK // 128
TBL = N * N
SC_TBL = 3 * TBL
SEG = SC_TBL // NTILES


def _count_body(src_hbm, dst_hbm, zeros_hbm, ones_hbm, out_hbm,
                src_v, dst_v, idx_v, ones_v, rb_v, table_sh, sem):
    del sem
    cid = lax.axis_index("c")
    sid = lax.axis_index("s")

    pltpu.sync_copy(zeros_hbm, rb_v)
    pltpu.sync_copy(ones_hbm, ones_v)
    pltpu.sync_copy(rb_v, table_sh.at[pl.ds(sid * SEG, SEG)])
    plsc.subcore_barrier()

    for rl in range(3):
        off = pl.multiple_of((cid * 3 + rl) * EPAD + sid * CHUNK, 128)
        pltpu.sync_copy(src_hbm.at[pl.ds(off, CHUNK)], src_v)
        pltpu.sync_copy(dst_hbm.at[pl.ds(off, CHUNK)], dst_v)
        base = rl * TBL

        def row_body(j, _):
            for u in range(8):
                s16 = src_v[pl.ds(j * 128 + u * 16, 16)]
                d16 = dst_v[pl.ds(j * 128 + u * 16, 16)]
                idx_v[pl.ds(j * 128 + u * 16, 16)] = d16 * N + s16 + base
            return 0

        lax.fori_loop(0, ROWS, row_body, 0)
        pltpu.sync_copy(ones_v, table_sh.at[idx_v], add=True)

    plsc.subcore_barrier()
    pltpu.sync_copy(table_sh.at[pl.ds(sid * SEG, SEG)], rb_v)
    out_off = pl.multiple_of(cid * SC_TBL + sid * SEG, 128)
    pltpu.sync_copy(rb_v, out_hbm.at[pl.ds(out_off, SEG)])


def _count_tables(src_all, dst_all):
    mesh = plsc.VectorSubcoreMesh(core_axis_name="c", subcore_axis_name="s")
    k = functools.partial(
        pl.kernel,
        out_type=jax.ShapeDtypeStruct((2 * SC_TBL,), jnp.float32),
        mesh=mesh,
        scratch_types=[
            pltpu.VMEM((CHUNK,), jnp.int32),
            pltpu.VMEM((CHUNK,), jnp.int32),
            pltpu.VMEM((CHUNK,), jnp.int32),
            pltpu.VMEM((CHUNK,), jnp.float32),
            pltpu.VMEM((SEG,), jnp.float32),
            pltpu.MemorySpace.VMEM_SHARED((SC_TBL + 128,), jnp.float32),
            pltpu.SemaphoreType.DMA,
        ],
    )(_count_body)
    flat = k(src_all, dst_all,
             jnp.zeros((SEG,), jnp.float32),
             jnp.ones((CHUNK,), jnp.float32))
    return flat.reshape(6 * N, N)


def _pad_rel(edges, r_local):
    pad = EPAD - E
    src = jnp.concatenate([edges[0], jnp.zeros((pad,), jnp.int32)])
    dst = jnp.concatenate(
        [edges[1], jnp.full((pad,), (3 - r_local) * N, jnp.int32)])
    return src, dst


def _mega_body(xt_ref, af_ref, ab_ref, wbig_ref, bf_ref, bb_ref,
               c_ref, wrel_ref, lw_ref, hb_ref, lng_ref, lnb_ref,
               cos_ref, sin_ref, p_ref, wtr_ref, wtp_ref, bt_ref,
               revf_ref, repf_ref, out_ref,
               xpf, xpb, tm, rpj, ppj):
    i = pl.program_id(0)

    @pl.when(i == 0)
    def _prep():
        xt = xt_ref[...]
        xpf[...] = jnp.dot(xt, af_ref[...],
                           preferred_element_type=jnp.float32) + bf_ref[...]
        xpb[...] = jnp.dot(xt, ab_ref[...],
                           preferred_element_type=jnp.float32) + bb_ref[...]

        def step(s, carry):
            h, c = carry
            xf = xpf[pl.ds(s * NB, NB), :]
            xb = xpb[pl.ds((T - 1 - s) * NB, NB), :]
            gates = jnp.dot(h, wbig_ref[...],
                            preferred_element_type=jnp.float32) + xf + xb
            ig = jax.nn.sigmoid(gates[:, 0:128])
            fg = jax.nn.sigmoid(gates[:, 128:256])
            gg = jnp.tanh(gates[:, 256:384])
            og = jax.nn.sigmoid(gates[:, 384:512])
            c2 = fg * c + ig * gg
            h2 = og * jnp.tanh(c2)
            tm[s, :, 0:64] = h2[:, 0:64]
            tm[T - 1 - s, :, 64:128] = h2[:, 64:128]
            return h2, c2

        z = jnp.zeros((NB, HIDDEN), jnp.float32)
        lax.fori_loop(0, T, step, (z, z))

        lstm = tm[...]
        xrev = jnp.concatenate([lstm[:, b, :] for b in range(4)], axis=0)
        xrep = jnp.concatenate([lstm[:, b, :] for b in range(4, 8)], axis=0)

        def conv(r, xsrc):
            C = c_ref[pl.ds(r * N, N), :]
            agg = jnp.dot(C, xsrc, preferred_element_type=jnp.float32)
            deg = jnp.maximum(jnp.sum(C, axis=1, keepdims=True), 1.0)
            W = wrel_ref[pl.ds(r * HIDDEN, HIDDEN), :]
            return jnp.dot(agg / deg, W, preferred_element_type=jnp.float32)

        loop = lw_ref[...]
        h_rev = (conv(0, xrev) + conv(3, xrep) + conv(5, xrev)
                 + jnp.dot(xrev, loop, preferred_element_type=jnp.float32)
                 + hb_ref[...])
        h_rep = (conv(1, xrep) + conv(2, xrev) + conv(4, xrep)
                 + jnp.dot(xrep, loop, preferred_element_type=jnp.float32)
                 + hb_ref[...])
        h_rev = jnp.maximum(h_rev, 0.0)
        h_rep = jnp.maximum(h_rep, 0.0)

        def layernorm(v):
            mu = jnp.mean(v, axis=1, keepdims=True)
            var = jnp.mean((v - mu) ** 2, axis=1, keepdims=True)
            return ((v - mu) / jnp.sqrt(var + 1e-5) * lng_ref[...]
                    + lnb_ref[...])

        revf = layernorm(xrev + h_rev)
        repf = layernorm(xrep + h_rep)
        revf_ref[...] = revf
        repf_ref[...] = repf

        p = p_ref[...]
        rot_rev = revf * cos_ref[...] + jnp.dot(
            revf, p, preferred_element_type=jnp.float32) * sin_ref[...]
        rot_rep = repf * cos_ref[...] + jnp.dot(
            repf, p, preferred_element_type=jnp.float32) * sin_ref[...]
        rpj[...] = jnp.dot(rot_rev, wtr_ref[...],
                           preferred_element_type=jnp.float32) + bt_ref[...]
        ppj[...] = jnp.dot(rot_rep, wtp_ref[...],
                           preferred_element_type=jnp.float32)

    b = i // 4
    ib = i % 4
    a = rpj[pl.ds(b * T + ib * 32, 32), :]
    bm = ppj[pl.ds(b * T, T), :]
    out_ref[...] = jnp.maximum(a[None, :, None, :] + bm[None, None, :, :],
                               0.0)


def _rotary_consts():
    inv = 10000.0 ** (-2.0 * np.arange(HL, dtype=np.float64) / HIDDEN)
    emb = np.arange(T, dtype=np.float64)[:, None] * inv[None, :]
    cos_t = np.repeat(np.cos(emb), 2, axis=1).astype(np.float32)
    sin_t = np.repeat(np.sin(emb), 2, axis=1).astype(np.float32)
    cosf = np.tile(cos_t, (4, 1))
    sinf = np.tile(sin_t, (4, 1))
    pm = np.zeros((HIDDEN, HIDDEN), np.float32)
    for kk in range(HL):
        pm[2 * kk + 1, 2 * kk] = -1.0
        pm[2 * kk, 2 * kk + 1] = 1.0
    return jnp.asarray(cosf), jnp.asarray(sinf), jnp.asarray(pm)


def _run_tc(review_input, reply_input, W_ih_f, W_hh_f, b_f,
            W_ih_b, W_hh_b, b_b, counts, W_rel, loop_weight, h_bias,
            W_table, b_table, ln_g, ln_b):
    x = jnp.concatenate([review_input, reply_input], axis=0)
    xt = jnp.transpose(x, (1, 0, 2)).reshape(T * NB, INPUT_DIM)

    z64 = jnp.zeros((INPUT_DIM, 4, HL), jnp.float32)
    af = jnp.concatenate([W_ih_f.T.reshape(INPUT_DIM, 4, HL), z64],
                         axis=2).reshape(INPUT_DIM, 512)
    ab = jnp.concatenate([z64, W_ih_b.T.reshape(INPUT_DIM, 4, HL)],
                         axis=2).reshape(INPUT_DIM, 512)
    zh = jnp.zeros((HL, 4, HL), jnp.float32)
    wtop = jnp.concatenate([W_hh_f.T.reshape(HL, 4, HL), zh], axis=2)
    wbot = jnp.concatenate([zh, W_hh_b.T.reshape(HL, 4, HL)], axis=2)
    wbig = jnp.concatenate([wtop, wbot], axis=0).reshape(HIDDEN, 512)
    zb = jnp.zeros((4, HL), jnp.float32)
    bf = jnp.concatenate([b_f.reshape(4, HL), zb], axis=1).reshape(1, 512)
    bb = jnp.concatenate([zb, b_b.reshape(4, HL)], axis=1).reshape(1, 512)

    cosf, sinf, pm = _rotary_consts()
    full = lambda shape: pl.BlockSpec(shape, lambda i: (0,) * len(shape))
    revf, repf, table_feat = pl.pallas_call(
        _mega_body,
        grid=(16,),
        in_specs=[
            full((T * NB, INPUT_DIM)), full((INPUT_DIM, 512)),
            full((INPUT_DIM, 512)), full((HIDDEN, 512)),
            full((1, 512)), full((1, 512)),
            full((6 * N, N)), full((6 * HIDDEN, HIDDEN)),
            full((HIDDEN, HIDDEN)), full((1, HIDDEN)),
            full((1, HIDDEN)), full((1, HIDDEN)),
            full((N, HIDDEN)), full((N, HIDDEN)), full((HIDDEN, HIDDEN)),
            full((HIDDEN, HIDDEN)), full((HIDDEN, HIDDEN)),
            full((1, HIDDEN)),
        ],
        out_specs=[
            full((N, HIDDEN)), full((N, HIDDEN)),
            pl.BlockSpec((1, 32, T, HIDDEN),
                         lambda i: (i // 4, i % 4, 0, 0)),
        ],
        out_shape=[
            jax.ShapeDtypeStruct((N, HIDDEN), jnp.float32),
            jax.ShapeDtypeStruct((N, HIDDEN), jnp.float32),
            jax.ShapeDtypeStruct((4, T, T, HIDDEN), jnp.float32),
        ],
        scratch_shapes=[
            pltpu.VMEM((T * NB, 512), jnp.float32),
            pltpu.VMEM((T * NB, 512), jnp.float32),
            pltpu.VMEM((T, NB, HIDDEN), jnp.float32),
            pltpu.VMEM((N, HIDDEN), jnp.float32),
            pltpu.VMEM((N, HIDDEN), jnp.float32),
        ],
    )(xt, af, ab, wbig, bf, bb,
      counts, W_rel.reshape(6 * HIDDEN, HIDDEN),
      loop_weight, h_bias.reshape(1, HIDDEN),
      ln_g.reshape(1, HIDDEN), ln_b.reshape(1, HIDDEN),
      cosf, sinf, pm,
      W_table[:HIDDEN], W_table[HIDDEN:], b_table.reshape(1, HIDDEN))
    return revf, repf, table_feat


def kernel(review_input, reply_input, table_input, review_seq_lens,
           reply_seq_lens, v2v_edges, b2b_edges, v2b_edges, b2v_edges,
           sl_rev_edges, sl_rep_edges, W_ih_f, W_hh_f, b_f, W_ih_b,
           W_hh_b, b_b, W_rel, loop_weight, h_bias, W_table, b_table,
           ln_g, ln_b):
    rels = [v2v_edges, b2b_edges, v2b_edges, b2v_edges,
            sl_rep_edges, sl_rev_edges]
    srcs, dsts = zip(*[_pad_rel(e, r % 3) for r, e in enumerate(rels)])
    counts = _count_tables(jnp.concatenate(srcs), jnp.concatenate(dsts))

    revf, repf, table_feat = _run_tc(
        review_input, reply_input, W_ih_f, W_hh_f, b_f, W_ih_b, W_hh_b,
        b_b, counts, W_rel, loop_weight, h_bias, W_table, b_table,
        ln_g, ln_b)
    review_feat = revf.reshape(4, T, HIDDEN)
    reply_feat = repf.reshape(4, T, HIDDEN)
    return review_feat, reply_feat, table_feat


# speedup vs baseline: 1.0838x; 1.0838x over previous
import functools

import jax
import jax.numpy as jnp
import numpy as np
from jax import lax
from jax.experimental import pallas as pl
from jax.experimental.pallas import tpu as pltpu
from jax.experimental.pallas import tpu_sc as plsc

HIDDEN = 128
INPUT_DIM = 768
HL = 64
NB = 8
T = 128
N = 512
E = 100000
EPAD = 100352
NTILES = 16
CHUNK = EPAD // NTILES
ROWS = CHUNK // 128
TBL = N * N
SC_TBL = 3 * TBL
SEG = SC_TBL // NTILES


def _count_body(src_hbm, dst_hbm, zeros_hbm, ones_hbm, out_hbm,
                src_v, dst_v, idx_v, ones_v, rb_v, table_sh, sem):
    del sem
    cid = lax.axis_index("c")
    sid = lax.axis_index("s")

    pltpu.sync_copy(zeros_hbm, rb_v)
    pltpu.sync_copy(ones_hbm, ones_v)
    pltpu.sync_copy(rb_v, table_sh.at[pl.ds(sid * SEG, SEG)])
    plsc.subcore_barrier()

    for rl in range(3):
        off = pl.multiple_of((cid * 3 + rl) * EPAD + sid * CHUNK, 128)
        pltpu.sync_copy(src_hbm.at[pl.ds(off, CHUNK)], src_v)
        pltpu.sync_copy(dst_hbm.at[pl.ds(off, CHUNK)], dst_v)
        base = rl * TBL

        def row_body(j, _):
            for u in range(8):
                s16 = src_v[pl.ds(j * 128 + u * 16, 16)]
                d16 = dst_v[pl.ds(j * 128 + u * 16, 16)]
                idx_v[pl.ds(j * 128 + u * 16, 16)] = d16 * N + s16 + base
            return 0

        lax.fori_loop(0, ROWS, row_body, 0)
        pltpu.sync_copy(ones_v, table_sh.at[idx_v], add=True)

    plsc.subcore_barrier()
    pltpu.sync_copy(table_sh.at[pl.ds(sid * SEG, SEG)], rb_v)
    out_off = pl.multiple_of(cid * SC_TBL + sid * SEG, 128)
    pltpu.sync_copy(rb_v, out_hbm.at[pl.ds(out_off, SEG)])


def _count_tables(src_all, dst_all):
    mesh = plsc.VectorSubcoreMesh(core_axis_name="c", subcore_axis_name="s")
    k = functools.partial(
        pl.kernel,
        out_type=jax.ShapeDtypeStruct((2 * SC_TBL,), jnp.float32),
        mesh=mesh,
        scratch_types=[
            pltpu.VMEM((CHUNK,), jnp.int32),
            pltpu.VMEM((CHUNK,), jnp.int32),
            pltpu.VMEM((CHUNK,), jnp.int32),
            pltpu.VMEM((CHUNK,), jnp.float32),
            pltpu.VMEM((SEG,), jnp.float32),
            pltpu.MemorySpace.VMEM_SHARED((SC_TBL + 128,), jnp.float32),
            pltpu.SemaphoreType.DMA,
        ],
    )(_count_body)
    flat = k(src_all, dst_all,
             jnp.zeros((SEG,), jnp.float32),
             jnp.ones((CHUNK,), jnp.float32))
    return flat.reshape(6 * N, N)


def _pad_rel(edges, r_local):
    pad = EPAD - E
    src = jnp.concatenate([edges[0], jnp.zeros((pad,), jnp.int32)])
    dst = jnp.concatenate(
        [edges[1], jnp.full((pad,), (3 - r_local) * N, jnp.int32)])
    return src, dst


def _lstm_body(xt_ref, af_ref, ab_ref, wbig_ref, bf_ref, bb_ref,
               out_ref, xpf, xpb, tm):
    xt = xt_ref[...]
    xpf[...] = jnp.dot(xt, af_ref[...],
                       preferred_element_type=jnp.float32) + bf_ref[...]
    xpb[...] = jnp.dot(xt, ab_ref[...],
                       preferred_element_type=jnp.float32) + bb_ref[...]

    def step(s, carry):
        h, c = carry
        xf = xpf[pl.ds(s * NB, NB), :]
        xb = xpb[pl.ds((T - 1 - s) * NB, NB), :]
        gates = jnp.dot(h, wbig_ref[...],
                        preferred_element_type=jnp.float32) + xf + xb
        ig = jax.nn.sigmoid(gates[:, 0:128])
        fg = jax.nn.sigmoid(gates[:, 128:256])
        gg = jnp.tanh(gates[:, 256:384])
        og = jax.nn.sigmoid(gates[:, 384:512])
        c2 = fg * c + ig * gg
        h2 = og * jnp.tanh(c2)
        tm[s, :, 0:64] = h2[:, 0:64]
        tm[T - 1 - s, :, 64:128] = h2[:, 64:128]
        return h2, c2

    z = jnp.zeros((NB, HIDDEN), jnp.float32)
    lax.fori_loop(0, T, step, (z, z))

    lstm = tm[...]
    for b in range(NB):
        out_ref[pl.ds(b * T, T), :] = lstm[:, b, :]


def _run_lstm(review_input, reply_input, W_ih_f, W_hh_f, b_f,
              W_ih_b, W_hh_b, b_b):
    x = jnp.concatenate([review_input, reply_input], axis=0)
    xt = jnp.transpose(x, (1, 0, 2)).reshape(T * NB, INPUT_DIM)

    z64 = jnp.zeros((INPUT_DIM, 4, HL), jnp.float32)
    af = jnp.concatenate([W_ih_f.T.reshape(INPUT_DIM, 4, HL), z64],
                         axis=2).reshape(INPUT_DIM, 512)
    ab = jnp.concatenate([z64, W_ih_b.T.reshape(INPUT_DIM, 4, HL)],
                         axis=2).reshape(INPUT_DIM, 512)
    zh = jnp.zeros((HL, 4, HL), jnp.float32)
    wtop = jnp.concatenate([W_hh_f.T.reshape(HL, 4, HL), zh], axis=2)
    wbot = jnp.concatenate([zh, W_hh_b.T.reshape(HL, 4, HL)], axis=2)
    wbig = jnp.concatenate([wtop, wbot], axis=0).reshape(HIDDEN, 512)
    zb = jnp.zeros((4, HL), jnp.float32)
    bf = jnp.concatenate([b_f.reshape(4, HL), zb], axis=1).reshape(1, 512)
    bb = jnp.concatenate([zb, b_b.reshape(4, HL)], axis=1).reshape(1, 512)

    return pl.pallas_call(
        _lstm_body,
        out_shape=jax.ShapeDtypeStruct((T * NB, HIDDEN), jnp.float32),
        scratch_shapes=[
            pltpu.VMEM((T * NB, 512), jnp.float32),
            pltpu.VMEM((T * NB, 512), jnp.float32),
            pltpu.VMEM((T, NB, HIDDEN), jnp.float32),
        ],
    )(xt, af, ab, wbig, bf, bb)


def _fuse_body(xbm_ref, c_ref, wrel_ref, lw_ref, hb_ref, lng_ref, lnb_ref,
               cos_ref, sin_ref, p_ref, wtr_ref, wtp_ref, bt_ref,
               revf_ref, repf_ref, out_ref, rpj, ppj):
    i = pl.program_id(0)

    @pl.when(i == 0)
    def _prep():
        xrev = xbm_ref[pl.ds(0, N), :]
        xrep = xbm_ref[pl.ds(N, N), :]

        def conv(r, xsrc):
            C = c_ref[pl.ds(r * N, N), :]
            agg = jnp.dot(C, xsrc, preferred_element_type=jnp.float32)
            deg = jnp.maximum(jnp.sum(C, axis=1, keepdims=True), 1.0)
            W = wrel_ref[pl.ds(r * HIDDEN, HIDDEN), :]
            return jnp.dot(agg / deg, W, preferred_element_type=jnp.float32)

        loop = lw_ref[...]
        h_rev = (conv(0, xrev) + conv(3, xrep) + conv(5, xrev)
                 + jnp.dot(xrev, loop, preferred_element_type=jnp.float32)
                 + hb_ref[...])
        h_rep = (conv(1, xrep) + conv(2, xrev) + conv(4, xrep)
                 + jnp.dot(xrep, loop, preferred_element_type=jnp.float32)
                 + hb_ref[...])
        h_rev = jnp.maximum(h_rev, 0.0)
        h_rep = jnp.maximum(h_rep, 0.0)

        def layernorm(v):
            mu = jnp.mean(v, axis=1, keepdims=True)
            var = jnp.mean((v - mu) ** 2, axis=1, keepdims=True)
            return ((v - mu) / jnp.sqrt(var + 1e-5) * lng_ref[...]
                    + lnb_ref[...])

        revf = layernorm(xrev + h_rev)
        repf = layernorm(xrep + h_rep)
        revf_ref[...] = revf
        repf_ref[...] = repf

        p = p_ref[...]
        rot_rev = revf * cos_ref[...] + jnp.dot(
            revf, p, preferred_element_type=jnp.float32) * sin_ref[...]
        rot_rep = repf * cos_ref[...] + jnp.dot(
            repf, p, preferred_element_type=jnp.float32) * sin_ref[...]
        rpj[...] = jnp.dot(rot_rev, wtr_ref[...],
                           preferred_element_type=jnp.float32) + bt_ref[...]
        ppj[...] = jnp.dot(rot_rep, wtp_ref[...],
                           preferred_element_type=jnp.float32)

    b = i // 4
    ib = i % 4
    a = rpj[pl.ds(b * T + ib * 32, 32), :]
    bm = ppj[pl.ds(b * T, T), :]
    out_ref[...] = jnp.maximum(a[None, :, None, :] + bm[None, None, :, :],
                               0.0)


def _rotary_consts():
    inv = 10000.0 ** (-2.0 * np.arange(HL, dtype=np.float64) / HIDDEN)
    emb = np.arange(T, dtype=np.float64)[:, None] * inv[None, :]
    cos_t = np.repeat(np.cos(emb), 2, axis=1).astype(np.float32)
    sin_t = np.repeat(np.sin(emb), 2, axis=1).astype(np.float32)
    cosf = np.tile(cos_t, (4, 1))
    sinf = np.tile(sin_t, (4, 1))
    pm = np.zeros((HIDDEN, HIDDEN), np.float32)
    for kk in range(HL):
        pm[2 * kk + 1, 2 * kk] = -1.0
        pm[2 * kk, 2 * kk + 1] = 1.0
    return jnp.asarray(cosf), jnp.asarray(sinf), jnp.asarray(pm)


def _run_fuse(xbm, counts, W_rel, loop_weight, h_bias,
              W_table, b_table, ln_g, ln_b):
    cosf, sinf, pm = _rotary_consts()
    full = lambda shape: pl.BlockSpec(shape, lambda i: (0,) * len(shape))
    return pl.pallas_call(
        _fuse_body,
        grid=(16,),
        in_specs=[
            full((T * NB, HIDDEN)), full((6 * N, N)),
            full((6 * HIDDEN, HIDDEN)), full((HIDDEN, HIDDEN)),
            full((1, HIDDEN)), full((1, HIDDEN)), full((1, HIDDEN)),
            full((N, HIDDEN)), full((N, HIDDEN)), full((HIDDEN, HIDDEN)),
            full((HIDDEN, HIDDEN)), full((HIDDEN, HIDDEN)),
            full((1, HIDDEN)),
        ],
        out_specs=[
            full((N, HIDDEN)), full((N, HIDDEN)),
            pl.BlockSpec((1, 32, T, HIDDEN),
                         lambda i: (i // 4, i % 4, 0, 0)),
        ],
        out_shape=[
            jax.ShapeDtypeStruct((N, HIDDEN), jnp.float32),
            jax.ShapeDtypeStruct((N, HIDDEN), jnp.float32),
            jax.ShapeDtypeStruct((4, T, T, HIDDEN), jnp.float32),
        ],
        scratch_shapes=[
            pltpu.VMEM((N, HIDDEN), jnp.float32),
            pltpu.VMEM((N, HIDDEN), jnp.float32),
        ],
    )(xbm, counts, W_rel.reshape(6 * HIDDEN, HIDDEN),
      loop_weight, h_bias.reshape(1, HIDDEN),
      ln_g.reshape(1, HIDDEN), ln_b.reshape(1, HIDDEN),
      cosf, sinf, pm,
      W_table[:HIDDEN], W_table[HIDDEN:], b_table.reshape(1, HIDDEN))


def kernel(review_input, reply_input, table_input, review_seq_lens,
           reply_seq_lens, v2v_edges, b2b_edges, v2b_edges, b2v_edges,
           sl_rev_edges, sl_rep_edges, W_ih_f, W_hh_f, b_f, W_ih_b,
           W_hh_b, b_b, W_rel, loop_weight, h_bias, W_table, b_table,
           ln_g, ln_b):
    rels = [v2v_edges, b2b_edges, v2b_edges, b2v_edges,
            sl_rep_edges, sl_rev_edges]
    srcs, dsts = zip(*[_pad_rel(e, r % 3) for r, e in enumerate(rels)])
    counts = _count_tables(jnp.concatenate(srcs), jnp.concatenate(dsts))

    xbm = _run_lstm(review_input, reply_input,
                    W_ih_f, W_hh_f, b_f, W_ih_b, W_hh_b, b_b)

    revf, repf, table_feat = _run_fuse(
        xbm, counts, W_rel, loop_weight, h_bias, W_table, b_table,
        ln_g, ln_b)
    review_feat = revf.reshape(4, T, HIDDEN)
    reply_feat = repf.reshape(4, T, HIDDEN)
    return review_feat, reply_feat, table_feat


# speedup vs baseline: 1.0946x; 1.0100x over previous
import functools

import jax
import jax.numpy as jnp
import numpy as np
from jax import lax
from jax.experimental import pallas as pl
from jax.experimental.pallas import tpu as pltpu
from jax.experimental.pallas import tpu_sc as plsc

HIDDEN = 128
INPUT_DIM = 768
HL = 64
NB = 8
T = 128
N = 512
E = 100000
EPAD = 100352
NTILES = 16
CHUNK = EPAD // NTILES
ROWS = CHUNK // 128
TBL = N * N
SC_TBL = 3 * TBL
SEG = SC_TBL // NTILES


def _count_body(src_hbm, dst_hbm, zeros_hbm, ones_hbm, out_hbm,
                src_v, dst_v, idx_v, ones_v, rb_v, table_sh, sem):
    del sem
    cid = lax.axis_index("c")
    sid = lax.axis_index("s")

    pltpu.sync_copy(zeros_hbm, rb_v)
    pltpu.sync_copy(ones_hbm, ones_v)
    pltpu.sync_copy(rb_v, table_sh.at[pl.ds(sid * SEG, SEG)])
    plsc.subcore_barrier()

    for rl in range(3):
        off = pl.multiple_of((cid * 3 + rl) * EPAD + sid * CHUNK, 128)
        pltpu.sync_copy(src_hbm.at[pl.ds(off, CHUNK)], src_v)
        pltpu.sync_copy(dst_hbm.at[pl.ds(off, CHUNK)], dst_v)
        base = rl * TBL

        def row_body(j, _):
            for u in range(8):
                s16 = src_v[pl.ds(j * 128 + u * 16, 16)]
                d16 = dst_v[pl.ds(j * 128 + u * 16, 16)]
                idx_v[pl.ds(j * 128 + u * 16, 16)] = d16 * N + s16 + base
            return 0

        lax.fori_loop(0, ROWS, row_body, 0)
        pltpu.sync_copy(ones_v, table_sh.at[idx_v], add=True)

    plsc.subcore_barrier()
    pltpu.sync_copy(table_sh.at[pl.ds(sid * SEG, SEG)], rb_v)
    out_off = pl.multiple_of(cid * SC_TBL + sid * SEG, 128)
    pltpu.sync_copy(rb_v, out_hbm.at[pl.ds(out_off, SEG)])


def _count_tables(src_all, dst_all):
    mesh = plsc.VectorSubcoreMesh(core_axis_name="c", subcore_axis_name="s")
    k = functools.partial(
        pl.kernel,
        out_type=jax.ShapeDtypeStruct((2 * SC_TBL,), jnp.float32),
        mesh=mesh,
        scratch_types=[
            pltpu.VMEM((CHUNK,), jnp.int32),
            pltpu.VMEM((CHUNK,), jnp.int32),
            pltpu.VMEM((CHUNK,), jnp.int32),
            pltpu.VMEM((CHUNK,), jnp.float32),
            pltpu.VMEM((SEG,), jnp.float32),
            pltpu.MemorySpace.VMEM_SHARED((SC_TBL + 128,), jnp.float32),
            pltpu.SemaphoreType.DMA,
        ],
    )(_count_body)
    flat = k(src_all, dst_all,
             jnp.zeros((SEG,), jnp.float32),
             jnp.ones((CHUNK,), jnp.float32))
    return flat.reshape(6 * N, N)


def _pad_rel(edges, r_local):
    pad = EPAD - E
    src = jnp.concatenate([edges[0], jnp.zeros((pad,), jnp.int32)])
    dst = jnp.concatenate(
        [edges[1], jnp.full((pad,), (3 - r_local) * N, jnp.int32)])
    return src, dst


def _lstm_body(xt_ref, af_ref, ab_ref, wbig_ref, bf_ref, bb_ref,
               out_ref, xpf, xpb, tm):
    xt = xt_ref[...]
    xpf[...] = jnp.dot(xt, af_ref[...],
                       preferred_element_type=jnp.float32) + bf_ref[...]
    xpb[...] = jnp.dot(xt, ab_ref[...],
                       preferred_element_type=jnp.float32) + bb_ref[...]

    def step(s, carry):
        h, c = carry
        xf = xpf[pl.ds(s * NB, NB), :]
        xb = xpb[pl.ds((T - 1 - s) * NB, NB), :]
        gates = jnp.dot(h, wbig_ref[...],
                        preferred_element_type=jnp.float32) + xf + xb
        ig = jax.nn.sigmoid(gates[:, 0:128])
        fg = jax.nn.sigmoid(gates[:, 128:256])
        gg = jnp.tanh(gates[:, 256:384])
        og = jax.nn.sigmoid(gates[:, 384:512])
        c2 = fg * c + ig * gg
        h2 = og * jnp.tanh(c2)
        tm[s, :, 0:64] = h2[:, 0:64]
        tm[T - 1 - s, :, 64:128] = h2[:, 64:128]
        return h2, c2

    z = jnp.zeros((NB, HIDDEN), jnp.float32)
    lax.fori_loop(0, T, step, (z, z))

    lstm = tm[...]
    for b in range(NB):
        out_ref[pl.ds(b * T, T), :] = lstm[:, b, :]


def _run_lstm(review_input, reply_input, W_ih_f, W_hh_f, b_f,
              W_ih_b, W_hh_b, b_b):
    x = jnp.concatenate([review_input, reply_input], axis=0)
    xt = jnp.transpose(x, (1, 0, 2)).reshape(T * NB, INPUT_DIM)

    z64 = jnp.zeros((INPUT_DIM, 4, HL), jnp.float32)
    af = jnp.concatenate([W_ih_f.T.reshape(INPUT_DIM, 4, HL), z64],
                         axis=2).reshape(INPUT_DIM, 512)
    ab = jnp.concatenate([z64, W_ih_b.T.reshape(INPUT_DIM, 4, HL)],
                         axis=2).reshape(INPUT_DIM, 512)
    zh = jnp.zeros((HL, 4, HL), jnp.float32)
    wtop = jnp.concatenate([W_hh_f.T.reshape(HL, 4, HL), zh], axis=2)
    wbot = jnp.concatenate([zh, W_hh_b.T.reshape(HL, 4, HL)], axis=2)
    wbig = jnp.concatenate([wtop, wbot], axis=0).reshape(HIDDEN, 512)
    zb = jnp.zeros((4, HL), jnp.float32)
    bf = jnp.concatenate([b_f.reshape(4, HL), zb], axis=1).reshape(1, 512)
    bb = jnp.concatenate([zb, b_b.reshape(4, HL)], axis=1).reshape(1, 512)

    return pl.pallas_call(
        _lstm_body,
        out_shape=jax.ShapeDtypeStruct((T * NB, HIDDEN), jnp.float32),
        scratch_shapes=[
            pltpu.VMEM((T * NB, 512), jnp.float32),
            pltpu.VMEM((T * NB, 512), jnp.float32),
            pltpu.VMEM((T, NB, HIDDEN), jnp.float32),
        ],
    )(xt, af, ab, wbig, bf, bb)


def _fuse_body(xbm_ref, c_ref, wrel_ref, lw_ref, hb_ref, lng_ref, lnb_ref,
               cos_ref, sin_ref, p_ref, wtr_ref, wtp_ref, bt_ref,
               revf_ref, repf_ref, out_ref, rpj, ppj):
    i = pl.program_id(0)

    @pl.when(i == 0)
    def _prep():
        xrev = xbm_ref[pl.ds(0, N), :]
        xrep = xbm_ref[pl.ds(N, N), :]

        def conv(r, xsrc):
            C = c_ref[pl.ds(r * N, N), :]
            agg = jnp.dot(C, xsrc, preferred_element_type=jnp.float32)
            deg = jnp.maximum(jnp.sum(C, axis=1, keepdims=True), 1.0)
            W = wrel_ref[pl.ds(r * HIDDEN, HIDDEN), :]
            return jnp.dot(agg / deg, W, preferred_element_type=jnp.float32)

        loop = lw_ref[...]
        h_rev = (conv(0, xrev) + conv(3, xrep) + conv(5, xrev)
                 + jnp.dot(xrev, loop, preferred_element_type=jnp.float32)
                 + hb_ref[...])
        h_rep = (conv(1, xrep) + conv(2, xrev) + conv(4, xrep)
                 + jnp.dot(xrep, loop, preferred_element_type=jnp.float32)
                 + hb_ref[...])
        h_rev = jnp.maximum(h_rev, 0.0)
        h_rep = jnp.maximum(h_rep, 0.0)

        def layernorm(v):
            mu = jnp.mean(v, axis=1, keepdims=True)
            var = jnp.mean((v - mu) ** 2, axis=1, keepdims=True)
            return ((v - mu) / jnp.sqrt(var + 1e-5) * lng_ref[...]
                    + lnb_ref[...])

        revf = layernorm(xrev + h_rev)
        repf = layernorm(xrep + h_rep)
        revf_ref[...] = revf
        repf_ref[...] = repf

        p = p_ref[...]
        rot_rev = revf * cos_ref[...] + jnp.dot(
            revf, p, preferred_element_type=jnp.float32) * sin_ref[...]
        rot_rep = repf * cos_ref[...] + jnp.dot(
            repf, p, preferred_element_type=jnp.float32) * sin_ref[...]
        rpj[...] = jnp.dot(rot_rev, wtr_ref[...],
                           preferred_element_type=jnp.float32) + bt_ref[...]
        ppj[...] = jnp.dot(rot_rep, wtp_ref[...],
                           preferred_element_type=jnp.float32)

    a = rpj[pl.ds(i * T, T), :]
    bm = ppj[pl.ds(i * T, T), :]
    out_ref[...] = jnp.maximum(a[None, :, None, :] + bm[None, None, :, :],
                               0.0)


def _rotary_consts():
    inv = 10000.0 ** (-2.0 * np.arange(HL, dtype=np.float64) / HIDDEN)
    emb = np.arange(T, dtype=np.float64)[:, None] * inv[None, :]
    cos_t = np.repeat(np.cos(emb), 2, axis=1).astype(np.float32)
    sin_t = np.repeat(np.sin(emb), 2, axis=1).astype(np.float32)
    cosf = np.tile(cos_t, (4, 1))
    sinf = np.tile(sin_t, (4, 1))
    pm = np.zeros((HIDDEN, HIDDEN), np.float32)
    for kk in range(HL):
        pm[2 * kk + 1, 2 * kk] = -1.0
        pm[2 * kk, 2 * kk + 1] = 1.0
    return jnp.asarray(cosf), jnp.asarray(sinf), jnp.asarray(pm)


def _run_fuse(xbm, counts, W_rel, loop_weight, h_bias,
              W_table, b_table, ln_g, ln_b):
    cosf, sinf, pm = _rotary_consts()
    full = lambda shape: pl.BlockSpec(shape, lambda i: (0,) * len(shape))
    return pl.pallas_call(
        _fuse_body,
        grid=(4,),
        in_specs=[
            full((T * NB, HIDDEN)), full((6 * N, N)),
            full((6 * HIDDEN, HIDDEN)), full((HIDDEN, HIDDEN)),
            full((1, HIDDEN)), full((1, HIDDEN)), full((1, HIDDEN)),
            full((N, HIDDEN)), full((N, HIDDEN)), full((HIDDEN, HIDDEN)),
            full((HIDDEN, HIDDEN)), full((HIDDEN, HIDDEN)),
            full((1, HIDDEN)),
        ],
        out_specs=[
            full((N, HIDDEN)), full((N, HIDDEN)),
            pl.BlockSpec((1, T, T, HIDDEN),
                         lambda i: (i, 0, 0, 0)),
        ],
        out_shape=[
            jax.ShapeDtypeStruct((N, HIDDEN), jnp.float32),
            jax.ShapeDtypeStruct((N, HIDDEN), jnp.float32),
            jax.ShapeDtypeStruct((4, T, T, HIDDEN), jnp.float32),
        ],
        scratch_shapes=[
            pltpu.VMEM((N, HIDDEN), jnp.float32),
            pltpu.VMEM((N, HIDDEN), jnp.float32),
        ],
    )(xbm, counts, W_rel.reshape(6 * HIDDEN, HIDDEN),
      loop_weight, h_bias.reshape(1, HIDDEN),
      ln_g.reshape(1, HIDDEN), ln_b.reshape(1, HIDDEN),
      cosf, sinf, pm,
      W_table[:HIDDEN], W_table[HIDDEN:], b_table.reshape(1, HIDDEN))


def kernel(review_input, reply_input, table_input, review_seq_lens,
           reply_seq_lens, v2v_edges, b2b_edges, v2b_edges, b2v_edges,
           sl_rev_edges, sl_rep_edges, W_ih_f, W_hh_f, b_f, W_ih_b,
           W_hh_b, b_b, W_rel, loop_weight, h_bias, W_table, b_table,
           ln_g, ln_b):
    rels = [v2v_edges, b2b_edges, v2b_edges, b2v_edges,
            sl_rep_edges, sl_rev_edges]
    srcs, dsts = zip(*[_pad_rel(e, r % 3) for r, e in enumerate(rels)])
    counts = _count_tables(jnp.concatenate(srcs), jnp.concatenate(dsts))

    xbm = _run_lstm(review_input, reply_input,
                    W_ih_f, W_hh_f, b_f, W_ih_b, W_hh_b, b_b)

    revf, repf, table_feat = _run_fuse(
        xbm, counts, W_rel, loop_weight, h_bias, W_table, b_table,
        ln_g, ln_b)
    review_feat = revf.reshape(4, T, HIDDEN)
    reply_feat = repf.reshape(4, T, HIDDEN)
    return review_feat, reply_feat, table_feat


# speedup vs baseline: 1.0953x; 1.0007x over previous
import functools

import jax
import jax.numpy as jnp
import numpy as np
from jax import lax
from jax.experimental import pallas as pl
from jax.experimental.pallas import tpu as pltpu
from jax.experimental.pallas import tpu_sc as plsc

HIDDEN = 128
INPUT_DIM = 768
HL = 64
NB = 8
T = 128
N = 512
E = 100000
EPAD = 100352
NTILES = 16
CHUNK = EPAD // NTILES
ROWS = CHUNK // 128
TBL = N * N
NC = 2
RPC = 6 // NC
SC_TBL = RPC * TBL
SEG = SC_TBL // NTILES


def _count_body(src_hbm, dst_hbm, zeros_hbm, ones_hbm, out_hbm,
                src_v, dst_v, idx_v, ones_v, rb_v, table_sh, sem):
    del sem
    cid = lax.axis_index("c")
    sid = lax.axis_index("s")

    pltpu.sync_copy(zeros_hbm, rb_v)
    pltpu.sync_copy(ones_hbm, ones_v)
    pltpu.sync_copy(rb_v, table_sh.at[pl.ds(sid * SEG, SEG)])
    plsc.subcore_barrier()

    for rl in range(RPC):
        off = pl.multiple_of((cid * RPC + rl) * EPAD + sid * CHUNK, 128)
        pltpu.sync_copy(src_hbm.at[pl.ds(off, CHUNK)], src_v)
        pltpu.sync_copy(dst_hbm.at[pl.ds(off, CHUNK)], dst_v)
        base = rl * TBL

        def row_body(j, _):
            for u in range(8):
                s16 = src_v[pl.ds(j * 128 + u * 16, 16)]
                d16 = dst_v[pl.ds(j * 128 + u * 16, 16)]
                idx_v[pl.ds(j * 128 + u * 16, 16)] = d16 * N + s16 + base
            return 0

        lax.fori_loop(0, ROWS, row_body, 0)
        pltpu.sync_copy(ones_v, table_sh.at[idx_v], add=True)

    plsc.subcore_barrier()
    pltpu.sync_copy(table_sh.at[pl.ds(sid * SEG, SEG)], rb_v)
    out_off = pl.multiple_of(cid * SC_TBL + sid * SEG, 128)
    pltpu.sync_copy(rb_v, out_hbm.at[pl.ds(out_off, SEG)])


def _count_tables(src_all, dst_all):
    mesh = plsc.VectorSubcoreMesh(core_axis_name="c", subcore_axis_name="s",
                                  num_cores=NC)
    k = functools.partial(
        pl.kernel,
        out_type=jax.ShapeDtypeStruct((NC * SC_TBL,), jnp.float32),
        mesh=mesh,
        scratch_types=[
            pltpu.VMEM((CHUNK,), jnp.int32),
            pltpu.VMEM((CHUNK,), jnp.int32),
            pltpu.VMEM((CHUNK,), jnp.int32),
            pltpu.VMEM((CHUNK,), jnp.float32),
            pltpu.VMEM((SEG,), jnp.float32),
            pltpu.MemorySpace.VMEM_SHARED((SC_TBL + 128,), jnp.float32),
            pltpu.SemaphoreType.DMA,
        ],
    )(_count_body)
    flat = k(src_all, dst_all,
             jnp.zeros((SEG,), jnp.float32),
             jnp.ones((CHUNK,), jnp.float32))
    return flat.reshape(6 * N, N)


def _pad_rel(edges, r_local):
    pad = EPAD - E
    src = jnp.concatenate([edges[0], jnp.zeros((pad,), jnp.int32)])
    dst = jnp.concatenate(
        [edges[1], jnp.full((pad,), (RPC - r_local) * N, jnp.int32)])
    return src, dst


def _lstm_body(xt_ref, af_ref, ab_ref, wbig_ref, bf_ref, bb_ref,
               out_ref, xpf, xpb, tm):
    xt = xt_ref[...]
    xpf[...] = jnp.dot(xt, af_ref[...],
                       preferred_element_type=jnp.float32) + bf_ref[...]
    xpb[...] = jnp.dot(xt, ab_ref[...],
                       preferred_element_type=jnp.float32) + bb_ref[...]

    def step(s, carry):
        h, c = carry
        xf = xpf[pl.ds(s * NB, NB), :]
        xb = xpb[pl.ds((T - 1 - s) * NB, NB), :]
        gates = jnp.dot(h, wbig_ref[...],
                        preferred_element_type=jnp.float32) + xf + xb
        ig = jax.nn.sigmoid(gates[:, 0:128])
        fg = jax.nn.sigmoid(gates[:, 128:256])
        gg = jnp.tanh(gates[:, 256:384])
        og = jax.nn.sigmoid(gates[:, 384:512])
        c2 = fg * c + ig * gg
        h2 = og * jnp.tanh(c2)
        tm[s, :, 0:64] = h2[:, 0:64]
        tm[T - 1 - s, :, 64:128] = h2[:, 64:128]
        return h2, c2

    z = jnp.zeros((NB, HIDDEN), jnp.float32)
    lax.fori_loop(0, T, step, (z, z))

    lstm = tm[...]
    for b in range(NB):
        out_ref[pl.ds(b * T, T), :] = lstm[:, b, :]


def _run_lstm(review_input, reply_input, W_ih_f, W_hh_f, b_f,
              W_ih_b, W_hh_b, b_b):
    x = jnp.concatenate([review_input, reply_input], axis=0)
    xt = jnp.transpose(x, (1, 0, 2)).reshape(T * NB, INPUT_DIM)

    z64 = jnp.zeros((INPUT_DIM, 4, HL), jnp.float32)
    af = jnp.concatenate([W_ih_f.T.reshape(INPUT_DIM, 4, HL), z64],
                         axis=2).reshape(INPUT_DIM, 512)
    ab = jnp.concatenate([z64, W_ih_b.T.reshape(INPUT_DIM, 4, HL)],
                         axis=2).reshape(INPUT_DIM, 512)
    zh = jnp.zeros((HL, 4, HL), jnp.float32)
    wtop = jnp.concatenate([W_hh_f.T.reshape(HL, 4, HL), zh], axis=2)
    wbot = jnp.concatenate([zh, W_hh_b.T.reshape(HL, 4, HL)], axis=2)
    wbig = jnp.concatenate([wtop, wbot], axis=0).reshape(HIDDEN, 512)
    zb = jnp.zeros((4, HL), jnp.float32)
    bf = jnp.concatenate([b_f.reshape(4, HL), zb], axis=1).reshape(1, 512)
    bb = jnp.concatenate([zb, b_b.reshape(4, HL)], axis=1).reshape(1, 512)

    return pl.pallas_call(
        _lstm_body,
        out_shape=jax.ShapeDtypeStruct((T * NB, HIDDEN), jnp.float32),
        scratch_shapes=[
            pltpu.VMEM((T * NB, 512), jnp.float32),
            pltpu.VMEM((T * NB, 512), jnp.float32),
            pltpu.VMEM((T, NB, HIDDEN), jnp.float32),
        ],
    )(xt, af, ab, wbig, bf, bb)


def _fuse_body(xbm_ref, c_ref, wrel_ref, lw_ref, hb_ref, lng_ref, lnb_ref,
               cos_ref, sin_ref, p_ref, wtr_ref, wtp_ref, bt_ref,
               revf_ref, repf_ref, out_ref, rpj, ppj):
    i = pl.program_id(0)

    @pl.when(i == 0)
    def _prep():
        xrev = xbm_ref[pl.ds(0, N), :]
        xrep = xbm_ref[pl.ds(N, N), :]

        def conv(r, xsrc):
            C = c_ref[pl.ds(r * N, N), :]
            agg = jnp.dot(C, xsrc, preferred_element_type=jnp.float32)
            deg = jnp.maximum(jnp.sum(C, axis=1, keepdims=True), 1.0)
            W = wrel_ref[pl.ds(r * HIDDEN, HIDDEN), :]
            return jnp.dot(agg / deg, W, preferred_element_type=jnp.float32)

        loop = lw_ref[...]
        h_rev = (conv(0, xrev) + conv(3, xrep) + conv(5, xrev)
                 + jnp.dot(xrev, loop, preferred_element_type=jnp.float32)
                 + hb_ref[...])
        h_rep = (conv(1, xrep) + conv(2, xrev) + conv(4, xrep)
                 + jnp.dot(xrep, loop, preferred_element_type=jnp.float32)
                 + hb_ref[...])
        h_rev = jnp.maximum(h_rev, 0.0)
        h_rep = jnp.maximum(h_rep, 0.0)

        def layernorm(v):
            mu = jnp.mean(v, axis=1, keepdims=True)
            var = jnp.mean((v - mu) ** 2, axis=1, keepdims=True)
            return ((v - mu) / jnp.sqrt(var + 1e-5) * lng_ref[...]
                    + lnb_ref[...])

        revf = layernorm(xrev + h_rev)
        repf = layernorm(xrep + h_rep)
        revf_ref[...] = revf
        repf_ref[...] = repf

        p = p_ref[...]
        rot_rev = revf * cos_ref[...] + jnp.dot(
            revf, p, preferred_element_type=jnp.float32) * sin_ref[...]
        rot_rep = repf * cos_ref[...] + jnp.dot(
            repf, p, preferred_element_type=jnp.float32) * sin_ref[...]
        rpj[...] = jnp.dot(rot_rev, wtr_ref[...],
                           preferred_element_type=jnp.float32) + bt_ref[...]
        ppj[...] = jnp.dot(rot_rep, wtp_ref[...],
                           preferred_element_type=jnp.float32)

    a = rpj[pl.ds(i * T, T), :]
    bm = ppj[pl.ds(i * T, T), :]
    out_ref[...] = jnp.maximum(a[None, :, None, :] + bm[None, None, :, :],
                               0.0)


def _rotary_consts():
    inv = 10000.0 ** (-2.0 * np.arange(HL, dtype=np.float64) / HIDDEN)
    emb = np.arange(T, dtype=np.float64)[:, None] * inv[None, :]
    cos_t = np.repeat(np.cos(emb), 2, axis=1).astype(np.float32)
    sin_t = np.repeat(np.sin(emb), 2, axis=1).astype(np.float32)
    cosf = np.tile(cos_t, (4, 1))
    sinf = np.tile(sin_t, (4, 1))
    pm = np.zeros((HIDDEN, HIDDEN), np.float32)
    for kk in range(HL):
        pm[2 * kk + 1, 2 * kk] = -1.0
        pm[2 * kk, 2 * kk + 1] = 1.0
    return jnp.asarray(cosf), jnp.asarray(sinf), jnp.asarray(pm)


def _run_fuse(xbm, counts, W_rel, loop_weight, h_bias,
              W_table, b_table, ln_g, ln_b):
    cosf, sinf, pm = _rotary_consts()
    full = lambda shape: pl.BlockSpec(shape, lambda i: (0,) * len(shape))
    return pl.pallas_call(
        _fuse_body,
        grid=(4,),
        in_specs=[
            full((T * NB, HIDDEN)), full((6 * N, N)),
            full((6 * HIDDEN, HIDDEN)), full((HIDDEN, HIDDEN)),
            full((1, HIDDEN)), full((1, HIDDEN)), full((1, HIDDEN)),
            full((N, HIDDEN)), full((N, HIDDEN)), full((HIDDEN, HIDDEN)),
            full((HIDDEN, HIDDEN)), full((HIDDEN, HIDDEN)),
            full((1, HIDDEN)),
        ],
        out_specs=[
            full((N, HIDDEN)), full((N, HIDDEN)),
            pl.BlockSpec((1, T, T, HIDDEN),
                         lambda i: (i, 0, 0, 0)),
        ],
        out_shape=[
            jax.ShapeDtypeStruct((N, HIDDEN), jnp.float32),
            jax.ShapeDtypeStruct((N, HIDDEN), jnp.float32),
            jax.ShapeDtypeStruct((4, T, T, HIDDEN), jnp.float32),
        ],
        scratch_shapes=[
            pltpu.VMEM((N, HIDDEN), jnp.float32),
            pltpu.VMEM((N, HIDDEN), jnp.float32),
        ],
    )(xbm, counts, W_rel.reshape(6 * HIDDEN, HIDDEN),
      loop_weight, h_bias.reshape(1, HIDDEN),
      ln_g.reshape(1, HIDDEN), ln_b.reshape(1, HIDDEN),
      cosf, sinf, pm,
      W_table[:HIDDEN], W_table[HIDDEN:], b_table.reshape(1, HIDDEN))


def kernel(review_input, reply_input, table_input, review_seq_lens,
           reply_seq_lens, v2v_edges, b2b_edges, v2b_edges, b2v_edges,
           sl_rev_edges, sl_rep_edges, W_ih_f, W_hh_f, b_f, W_ih_b,
           W_hh_b, b_b, W_rel, loop_weight, h_bias, W_table, b_table,
           ln_g, ln_b):
    rels = [v2v_edges, b2b_edges, v2b_edges, b2v_edges,
            sl_rep_edges, sl_rev_edges]
    srcs, dsts = zip(*[_pad_rel(e, r % RPC) for r, e in enumerate(rels)])
    counts = _count_tables(jnp.concatenate(srcs), jnp.concatenate(dsts))

    xbm = _run_lstm(review_input, reply_input,
                    W_ih_f, W_hh_f, b_f, W_ih_b, W_hh_b, b_b)

    revf, repf, table_feat = _run_fuse(
        xbm, counts, W_rel, loop_weight, h_bias, W_table, b_table,
        ln_g, ln_b)
    review_feat = revf.reshape(4, T, HIDDEN)
    reply_feat = repf.reshape(4, T, HIDDEN)
    return review_feat, reply_feat, table_feat
